# Initial kernel scaffold; baseline (speedup 1.0000x reference)
import jax
import jax.numpy as jnp
from jax.experimental import pallas as pl


def kernel(ufea, vfea, edge_index, uv_vals, vu_vals, gc1_W, gc1_b, gc2_W, gc2_b, user_union_W, user_union_b, item_union_W, item_union_b):
    raise NotImplementedError("write your pallas kernel here")


# speedup vs baseline: 2.8896x; 2.8896x over previous
import functools

import jax
import jax.numpy as jnp
from jax import lax
from jax.experimental import pallas as pl
from jax.experimental.pallas import tpu as pltpu
from jax.experimental.pallas import tpu_sc as plsc

N = 10000
D = 128
E = 320000
ALPHA = 0.1

NC = 2
NS = 16
CHUNK = 128
CPW = -(-E // (NC * NS * CHUNK))
E_PAD = NC * NS * CHUNK * CPW
NP = 10240
ROWS_PER_SUB = NP // NS
ZROWS = 128


def _segsum_kernel(table, sidx, didx, vals, out,
                   zbuf, idxs_v, idxd_v, vals_v, rows_v, sem, acc):
    c = lax.axis_index("c")
    s = lax.axis_index("s")
    wid = s * NC + c

    def _zrow(i, _):
        for j in range(D // 16):
            zbuf[i, pl.ds(j * 16, 16)] = jnp.zeros((16,), jnp.float32)
        return 0
    lax.fori_loop(0, ZROWS, _zrow, 0)
    for r in range(ROWS_PER_SUB // ZROWS):
        pltpu.sync_copy(zbuf, acc.at[pl.ds(s * ROWS_PER_SUB + r * ZROWS, ZROWS)])
    plsc.subcore_barrier()

    def _chunk(t, _):
        base = (wid * CPW + t) * CHUNK
        pltpu.sync_copy(sidx.at[pl.ds(base, CHUNK)], idxs_v)
        pltpu.sync_copy(didx.at[pl.ds(base, CHUNK)], idxd_v)
        pltpu.sync_copy(vals.at[pl.ds(base, CHUNK)], vals_v)
        pltpu.async_copy(table.at[idxs_v], rows_v, sem).wait()

        def _scale(g, _):
            vv = vals_v[pl.ds(g * 16, 16)]
            for k in range(16):
                sv = lax.broadcast(vv[k], (16,))
                row = g * 16 + k
                for j in range(D // 16):
                    sl = pl.ds(j * 16, 16)
                    rows_v[row, sl] = rows_v[row, sl] * sv
            return 0
        lax.fori_loop(0, CHUNK // 16, _scale, 0)

        pltpu.sync_copy(rows_v, acc.at[idxd_v], add=True)
        return 0
    lax.fori_loop(0, CPW, _chunk, 0)
    plsc.subcore_barrier()

    pltpu.sync_copy(acc.at[pl.ds(s * ROWS_PER_SUB, ROWS_PER_SUB)],
                    out.at[c, pl.ds(s * ROWS_PER_SUB, ROWS_PER_SUB)])


@jax.jit
def _segsum(table, sidx, didx, vals):
    mesh = plsc.VectorSubcoreMesh(core_axis_name="c", subcore_axis_name="s")
    f = functools.partial(
        pl.kernel,
        mesh=mesh,
        out_type=jax.ShapeDtypeStruct((NC, NP, D), jnp.float32),
        scratch_types=[
            pltpu.VMEM((ZROWS, D), jnp.float32),
            pltpu.VMEM((CHUNK,), jnp.int32),
            pltpu.VMEM((CHUNK,), jnp.int32),
            pltpu.VMEM((CHUNK,), jnp.float32),
            pltpu.VMEM((CHUNK, D), jnp.float32),
            pltpu.SemaphoreType.DMA,
            pltpu.VMEM_SHARED((NP, D), jnp.float32),
        ],
    )(_segsum_kernel)
    return f(table, sidx, didx, vals)[:, :N]


def _ho_body(p_ref, w_ref, b_ref, o_ref):
    x = p_ref[0] + p_ref[1]
    y = jnp.dot(x, w_ref[...], preferred_element_type=jnp.float32) + b_ref[...]
    o_ref[...] = jnp.where(y >= 0, y, ALPHA * y)


@jax.jit
def _ho(partials, w, b):
    blk = 1000
    grid = N // blk
    return pl.pallas_call(
        _ho_body,
        grid=(grid,),
        in_specs=[
            pl.BlockSpec((NC, blk, D), lambda i: (0, i, 0)),
            pl.BlockSpec((D, D), lambda i: (0, 0)),
            pl.BlockSpec((1, D), lambda i: (0, 0)),
        ],
        out_specs=pl.BlockSpec((blk, D), lambda i: (i, 0)),
        out_shape=jax.ShapeDtypeStruct((N, D), jnp.float32),
    )(partials, w, b.reshape(1, D))


def _final_body(p_ref, fea_ref, wt_ref, wb_ref, b_ref, o_ref):
    x = p_ref[0] + p_ref[1]
    x = jnp.where(x >= 0, x, ALPHA * x)
    y = (jnp.dot(x, wt_ref[...], preferred_element_type=jnp.float32)
         + jnp.dot(fea_ref[...], wb_ref[...], preferred_element_type=jnp.float32)
         + b_ref[...])
    o_ref[...] = jnp.maximum(y, 0.0)


@jax.jit
def _final(partials, fea, w_top, w_bot, b):
    blk = 1000
    grid = N // blk
    return pl.pallas_call(
        _final_body,
        grid=(grid,),
        in_specs=[
            pl.BlockSpec((NC, blk, D), lambda i: (0, i, 0)),
            pl.BlockSpec((blk, D), lambda i: (i, 0)),
            pl.BlockSpec((D, D), lambda i: (0, 0)),
            pl.BlockSpec((D, D), lambda i: (0, 0)),
            pl.BlockSpec((1, D), lambda i: (0, 0)),
        ],
        out_specs=pl.BlockSpec((blk, D), lambda i: (i, 0)),
        out_shape=jax.ShapeDtypeStruct((N, D), jnp.float32),
    )(partials, fea, w_top, w_bot, b.reshape(1, D))


def kernel(ufea, vfea, edge_index, uv_vals, vu_vals, gc1_W, gc1_b, gc2_W,
           gc2_b, user_union_W, user_union_b, item_union_W, item_union_b):
    u_idx = edge_index[0].astype(jnp.int32)
    v_idx = edge_index[1].astype(jnp.int32)
    pad = E_PAD - E
    u_pad = jnp.concatenate([u_idx, jnp.zeros((pad,), jnp.int32)])
    v_pad = jnp.concatenate([v_idx, jnp.zeros((pad,), jnp.int32)])
    uv_pad = jnp.concatenate([uv_vals, jnp.zeros((pad,), jnp.float32)])
    vu_pad = jnp.concatenate([vu_vals, jnp.zeros((pad,), jnp.float32)])

    s1 = _segsum(ufea, u_pad, v_pad, vu_pad)
    s2 = _segsum(vfea, v_pad, u_pad, uv_pad)
    user_ho = _ho(s1, gc1_W, gc1_b)
    item_ho = _ho(s2, gc2_W, gc2_b)

    s3 = _segsum(user_ho, v_pad, u_pad, uv_pad)
    s4 = _segsum(item_ho, u_pad, v_pad, vu_pad)

    user = _final(s3, ufea, user_union_W[:D], user_union_W[D:], user_union_b)
    item = _final(s4, vfea, item_union_W[:D], item_union_W[D:], item_union_b)
    return (user, item)
